# baseline (speedup 1.0000x reference)
import functools

import jax
import jax.numpy as jnp
from jax import lax
from jax.experimental import pallas as pl
from jax.experimental.pallas import tpu as pltpu
from jax.experimental.pallas import tpu_sc as plsc

E = 320000
T = 640000
D = 128
NB = 20

BE = 512
BT = 1024


def _silu(x):
    return x * jax.nn.sigmoid(x)


def _pq_body(x_ref, w_ref, p_ref, q_ref):
    x = x_ref[...]
    w = w_ref[...]
    r = jnp.dot(x, w, preferred_element_type=jnp.float32)
    p_ref[...] = r[:, :D]
    q_ref[...] = r[:, D:]


def _pq_matmul(edge_attr, w_ab):
    return pl.pallas_call(
        _pq_body,
        grid=(E // BE,),
        in_specs=[
            pl.BlockSpec((BE, D), lambda i: (i, 0)),
            pl.BlockSpec((D, 2 * D), lambda i: (0, 0)),
        ],
        out_specs=[
            pl.BlockSpec((BE, D), lambda i: (i, 0)),
            pl.BlockSpec((BE, D), lambda i: (i, 0)),
        ],
        out_shape=[
            jax.ShapeDtypeStruct((E, D), jnp.float32),
            jax.ShapeDtypeStruct((E, D), jnp.float32),
        ],
    )(edge_attr, w_ab)


def _angle_body(g_ref, wa1_ref, ba1_ref, wc_ref, cc_ref, a_ref):
    d2i = g_ref[0, :]
    d2k = g_ref[1, :]
    dot = g_ref[2, :]
    li = jnp.maximum(jnp.sqrt(d2i), 1e-6)
    lk = jnp.maximum(jnp.sqrt(d2k), 1e-6)
    cos = jnp.clip(dot / (li * lk), -1.0, 1.0)
    wa1 = wa1_ref[...]
    h = (li[:, None] * wa1[0:1, :] + lk[:, None] * wa1[1:2, :]
         + cos[:, None] * wa1[2:3, :] + ba1_ref[...])
    h = _silu(h)
    a_ref[...] = jnp.dot(h, wc_ref[...],
                         preferred_element_type=jnp.float32) + cc_ref[...]


def _angle_mlp(g3, wa1, ba1, wc, cc):
    return pl.pallas_call(
        _angle_body,
        grid=(T // BT,),
        in_specs=[
            pl.BlockSpec((3, BT), lambda i: (0, i)),
            pl.BlockSpec((3, NB), lambda i: (0, 0)),
            pl.BlockSpec((1, NB), lambda i: (0, 0)),
            pl.BlockSpec((NB, D), lambda i: (0, 0)),
            pl.BlockSpec((1, D), lambda i: (0, 0)),
        ],
        out_specs=pl.BlockSpec((BT, D), lambda i: (i, 0)),
        out_shape=jax.ShapeDtypeStruct((T, D), jnp.float32),
    )(g3, wa1, ba1, wc, cc)


def _final_body(s_ref, cnt_ref, m_ref, b2u_ref, bu_ref, o_ref):
    s = s_ref[...]
    cnt = cnt_ref[...]
    o_ref[...] = (jnp.dot(s, m_ref[...], preferred_element_type=jnp.float32)
                  + cnt * b2u_ref[...] + bu_ref[...])


def _final_matmul(s, cnt2d, m, b2u, bu):
    return pl.pallas_call(
        _final_body,
        grid=(E // BE,),
        in_specs=[
            pl.BlockSpec((BE, D), lambda i: (i, 0)),
            pl.BlockSpec((BE, 1), lambda i: (i, 0)),
            pl.BlockSpec((D, D), lambda i: (0, 0)),
            pl.BlockSpec((1, D), lambda i: (0, 0)),
            pl.BlockSpec((1, D), lambda i: (0, 0)),
        ],
        out_specs=pl.BlockSpec((BE, D), lambda i: (i, 0)),
        out_shape=jax.ShapeDtypeStruct((E, D), jnp.float32),
    )(s, cnt2d, m, b2u, bu)


NW = 32
TPW = T // NW
GCH = 4000


def _geo_body(eij_hbm, eik_hbm, vx_hbm, vy_hbm, vz_hbm,
              g0_hbm, g1_hbm, g2_hbm,
              iij, iik, xij, yij, zij, xik, yik, zik, g0, g1, g2, sem):
    wid = lax.axis_index("s") * 2 + lax.axis_index("c")

    for ch in range(TPW // GCH):
        base = wid * TPW + ch * GCH
        pltpu.sync_copy(eij_hbm.at[pl.ds(base, GCH)], iij)
        pltpu.sync_copy(eik_hbm.at[pl.ds(base, GCH)], iik)
        cps = [pltpu.async_copy(vx_hbm.at[iij], xij, sem),
               pltpu.async_copy(vy_hbm.at[iij], yij, sem),
               pltpu.async_copy(vz_hbm.at[iij], zij, sem),
               pltpu.async_copy(vx_hbm.at[iik], xik, sem),
               pltpu.async_copy(vy_hbm.at[iik], yik, sem),
               pltpu.async_copy(vz_hbm.at[iik], zik, sem)]
        for cp in cps:
            cp.wait()

        def body(i, carry):
            s = pl.ds(i * 16, 16)
            a, b, c = xij[s], yij[s], zij[s]
            d, e, f = xik[s], yik[s], zik[s]
            g0[s] = a * a + b * b + c * c
            g1[s] = d * d + e * e + f * f
            g2[s] = a * d + b * e + c * f
            return carry

        lax.fori_loop(0, GCH // 16, body, 0)
        pltpu.sync_copy(g0, g0_hbm.at[pl.ds(base, GCH)])
        pltpu.sync_copy(g1, g1_hbm.at[pl.ds(base, GCH)])
        pltpu.sync_copy(g2, g2_hbm.at[pl.ds(base, GCH)])


def _geometry(e_ij, e_ik, vx, vy, vz):
    mesh = plsc.VectorSubcoreMesh(core_axis_name="c", subcore_axis_name="s")
    f = pl.kernel(
        _geo_body,
        out_type=(jax.ShapeDtypeStruct((T,), jnp.float32),
                  jax.ShapeDtypeStruct((T,), jnp.float32),
                  jax.ShapeDtypeStruct((T,), jnp.float32)),
        mesh=mesh,
        scratch_types=(
            [pltpu.VMEM((GCH,), jnp.int32)] * 2
            + [pltpu.VMEM((GCH,), jnp.float32)] * 9
            + [pltpu.SemaphoreType.DMA]
        ),
    )
    return f(e_ij, e_ik, vx, vy, vz)


BROW = 8192
NBK = 40
EP = BROW * NBK
ACC = BROW + 16
DUMMY = BROW
SCH = 2000
CAP = SCH // 16
SUBSZ = 16 * CAP + 32
COMPSZ = SCH + 80
TPS = T // 16
CSUB = 96


def _silu16(x):
    return x / (1.0 + jnp.exp(-x))


def _scatter_body(eij_hbm, eik_hbm, p_hbm, q_hbm, a_hbm,
                  s_hbm, cnt_hbm,
                  ebij, ebik, tsub, ijsub, iksub, tcomp, ijcomp, ikcomp,
                  pbuf, qbuf, abuf, lidx, ones_c, zrow, zcnt,
                  accum, cntacc, sem):
    cid = lax.axis_index("c")
    tid = lax.axis_index("s")
    lane = lax.iota(jnp.int32, 16)
    lanecap = lane * CAP
    zvec = jnp.zeros((16,), jnp.int32)

    def init_body(i, carry):
        sl = pl.ds(i * 16, 16)
        tsub[sl] = zvec
        ijsub[sl] = zvec
        iksub[sl] = zvec
        return carry
    lax.fori_loop(0, SUBSZ // 16, init_body, 0)

    def init_body2(i, carry):
        sl = pl.ds(i * 16, 16)
        tcomp[sl] = zvec
        ijcomp[sl] = zvec
        ikcomp[sl] = zvec
        return carry
    lax.fori_loop(0, COMPSZ // 16, init_body2, 0)

    def zrow_body(i, carry):
        for k in range(D // 16):
            zrow[i, pl.ds(k * 16, 16)] = jnp.zeros((16,), jnp.float32)
        return carry
    lax.fori_loop(0, 32, zrow_body, 0)

    def zcnt_body(i, carry):
        zcnt[pl.ds(i * 16, 16)] = jnp.zeros((16,), jnp.float32)
        return carry
    lax.fori_loop(0, 512 // 16, zcnt_body, 0)

    def ones_body(i, carry):
        ones_c[pl.ds(i * 16, 16)] = jnp.ones((16,), jnp.float32)
        return carry
    lax.fori_loop(0, CSUB // 16, ones_body, 0)

    def bucket_body(j, carry0):
        p = 2 * j + cid
        lo = p * BROW

        for r in range(16):
            pltpu.sync_copy(zrow, accum.at[pl.ds(tid * 512 + r * 32, 32)])
        pltpu.sync_copy(zcnt, cntacc.at[pl.ds(tid * 512, 512)])

        @pl.when(tid == 0)
        def _():
            pltpu.sync_copy(zcnt.at[pl.ds(0, 16)],
                            cntacc.at[pl.ds(BROW, 16)])
            pltpu.sync_copy(zrow.at[pl.ds(0, 16)],
                            accum.at[pl.ds(BROW, 16)])

        plsc.subcore_barrier()

        def chunk_body(sc, carry1):
            sbase = tid * TPS + sc * SCH
            pltpu.sync_copy(eij_hbm.at[pl.ds(sbase, SCH)], ebij)
            pltpu.sync_copy(eik_hbm.at[pl.ds(sbase, SCH)], ebik)

            def cbody(i, w_vec):
                sl = pl.ds(i * 16, 16)
                ev = ebij[sl]
                kv = ebik[sl]
                m = (ev >= lo) & (ev < lo + BROW)
                mi = jnp.where(m, 1, 0)
                tv = lane + (sbase + i * 16)
                pos = jnp.where(m, lanecap + w_vec, (16 * CAP + 16) + lane)
                plsc.store_scatter(tsub, [pos], tv)
                plsc.store_scatter(ijsub, [pos], ev)
                plsc.store_scatter(iksub, [pos], kv)
                return w_vec + mi

            counts = lax.fori_loop(0, SCH // 16, cbody, zvec)

            m_tot = jnp.int32(0)
            for l in range(16):
                cl = counts[l]
                ng = (cl + 15) // 16
                mt = m_tot

                def copyg(g, carry2, _l=l, _mt=mt):
                    d = pl.ds(_mt + g * 16, 16)
                    s = pl.ds(_l * CAP + g * 16, 16)
                    tcomp[d] = tsub[s]
                    ijcomp[d] = ijsub[s]
                    ikcomp[d] = iksub[s]
                    return carry2
                lax.fori_loop(0, ng, copyg, 0)
                m_tot = m_tot + cl

            nsub = (m_tot + CSUB - 1) // CSUB

            def pbody(u, carry2):
                off = u * CSUB
                ij_c = ijcomp.at[pl.ds(off, CSUB)]
                ik_c = ikcomp.at[pl.ds(off, CSUB)]
                t_c = tcomp.at[pl.ds(off, CSUB)]
                cp1 = pltpu.async_copy(p_hbm.at[ij_c], pbuf, sem)
                cp2 = pltpu.async_copy(q_hbm.at[ik_c], qbuf, sem)
                cp3 = pltpu.async_copy(a_hbm.at[t_c], abuf, sem)
                cp1.wait()
                cp2.wait()
                cp3.wait()

                def zbody(r, carry3):
                    for k in range(D // 16):
                        sl = pl.ds(k * 16, 16)
                        x = pbuf[r, sl] + qbuf[r, sl] + abuf[r, sl]
                        pbuf[r, sl] = _silu16(x)
                    return carry3
                lax.fori_loop(0, CSUB, zbody, 0)

                def lbody(i, carry3):
                    sl = pl.ds(i * 16, 16)
                    e = ijcomp[pl.ds(off + i * 16, 16)]
                    posv = (off + i * 16) + lane
                    valid = posv < m_tot
                    lidx[sl] = jnp.where(valid, e - lo, DUMMY)
                    return carry3
                lax.fori_loop(0, CSUB // 16, lbody, 0)

                pltpu.sync_copy(pbuf, accum.at[lidx], add=True)
                pltpu.sync_copy(ones_c, cntacc.at[lidx], add=True)
                return carry2

            lax.fori_loop(0, nsub, pbody, 0)
            return carry1

        lax.fori_loop(0, TPS // SCH, chunk_body, 0)
        plsc.subcore_barrier()

        pltpu.sync_copy(accum.at[pl.ds(tid * 512, 512)],
                        s_hbm.at[pl.ds(lo + tid * 512, 512)])

        @pl.when(tid == 0)
        def _():
            pltpu.sync_copy(cntacc.at[pl.ds(0, BROW)],
                            cnt_hbm.at[pl.ds(lo, BROW)])
        return carry0

    lax.fori_loop(0, NBK // 2, bucket_body, 0)


def _gather_silu_scatter(e_ij, e_ik, p_arr, q_arr, a_arr):
    mesh = plsc.VectorSubcoreMesh(core_axis_name="c", subcore_axis_name="s")
    f = pl.kernel(
        _scatter_body,
        out_type=(jax.ShapeDtypeStruct((EP, D), jnp.float32),
                  jax.ShapeDtypeStruct((EP,), jnp.float32)),
        mesh=mesh,
        compiler_params=pltpu.CompilerParams(needs_layout_passes=False),
        scratch_types=[
            pltpu.VMEM((SCH,), jnp.int32),
            pltpu.VMEM((SCH,), jnp.int32),
            pltpu.VMEM((SUBSZ,), jnp.int32),
            pltpu.VMEM((SUBSZ,), jnp.int32),
            pltpu.VMEM((SUBSZ,), jnp.int32),
            pltpu.VMEM((COMPSZ,), jnp.int32),
            pltpu.VMEM((COMPSZ,), jnp.int32),
            pltpu.VMEM((COMPSZ,), jnp.int32),
            pltpu.VMEM((CSUB, D), jnp.float32),
            pltpu.VMEM((CSUB, D), jnp.float32),
            pltpu.VMEM((CSUB, D), jnp.float32),
            pltpu.VMEM((CSUB,), jnp.int32),
            pltpu.VMEM((CSUB,), jnp.float32),
            pltpu.VMEM((32, D), jnp.float32),
            pltpu.VMEM((512,), jnp.float32),
            pltpu.VMEM_SHARED((ACC, D), jnp.float32),
            pltpu.VMEM_SHARED((ACC,), jnp.float32),
            pltpu.SemaphoreType.DMA,
        ],
    )
    return f(e_ij, e_ik, p_arr, q_arr, a_arr)


def kernel(edge_attr, three_body_indices, three_body_edge_indices,
           edge_vectors, Wa1, ba1, Wa2, ba2, W1, b1, W2, b2, Wu, bu):
    del three_body_indices
    e_ij = three_body_edge_indices[:, 0]
    e_ik = three_body_edge_indices[:, 1]

    w_ab = W1[:2 * D, :]
    w_ab = jnp.concatenate([W1[:D, :], W1[D:2 * D, :]], axis=1)
    wc = Wa2 @ W1[2 * D:, :]
    cc = (ba2 @ W1[2 * D:, :] + b1)[None, :]
    m_fold = W2 @ Wu
    b2u = (b2 @ Wu)[None, :]
    bu2 = bu[None, :]

    p_arr, q_arr = _pq_matmul(edge_attr, w_ab)

    vx = edge_vectors[:, 0]
    vy = edge_vectors[:, 1]
    vz = edge_vectors[:, 2]
    g0, g1, g2 = _geometry(e_ij, e_ik, vx, vy, vz)
    g3 = jnp.stack([g0, g1, g2], axis=0)

    a_arr = _angle_mlp(g3, Wa1, ba1[None, :], wc, cc)

    s_pad, cnt_pad = _gather_silu_scatter(e_ij, e_ik, p_arr, q_arr, a_arr)

    return _final_matmul(s_pad, cnt_pad[:, None], m_fold, b2u, bu2)


# speedup vs baseline: 4.8208x; 4.8208x over previous
import functools

import jax
import jax.numpy as jnp
from jax import lax
from jax.experimental import pallas as pl
from jax.experimental.pallas import tpu as pltpu
from jax.experimental.pallas import tpu_sc as plsc

E = 320000
T = 640000
D = 128
NB = 20

BE = 512
BT = 1024


def _silu(x):
    return x * jax.nn.sigmoid(x)


def _pq_body(x_ref, w_ref, p_ref, q_ref):
    x = x_ref[...]
    w = w_ref[...]
    r = jnp.dot(x, w, preferred_element_type=jnp.float32)
    p_ref[...] = r[:, :D]
    q_ref[...] = r[:, D:]


def _pq_matmul(edge_attr, w_ab):
    return pl.pallas_call(
        _pq_body,
        grid=(E // BE,),
        in_specs=[
            pl.BlockSpec((BE, D), lambda i: (i, 0)),
            pl.BlockSpec((D, 2 * D), lambda i: (0, 0)),
        ],
        out_specs=[
            pl.BlockSpec((BE, D), lambda i: (i, 0)),
            pl.BlockSpec((BE, D), lambda i: (i, 0)),
        ],
        out_shape=[
            jax.ShapeDtypeStruct((E, D), jnp.float32),
            jax.ShapeDtypeStruct((E, D), jnp.float32),
        ],
    )(edge_attr, w_ab)


def _angle_body(g_ref, wa1_ref, ba1_ref, wc_ref, cc_ref, a_ref):
    d2i = g_ref[0, :]
    d2k = g_ref[1, :]
    dot = g_ref[2, :]
    li = jnp.maximum(jnp.sqrt(d2i), 1e-6)
    lk = jnp.maximum(jnp.sqrt(d2k), 1e-6)
    cos = jnp.clip(dot / (li * lk), -1.0, 1.0)
    wa1 = wa1_ref[...]
    h = (li[:, None] * wa1[0:1, :] + lk[:, None] * wa1[1:2, :]
         + cos[:, None] * wa1[2:3, :] + ba1_ref[...])
    h = _silu(h)
    a_ref[...] = jnp.dot(h, wc_ref[...],
                         preferred_element_type=jnp.float32) + cc_ref[...]


def _angle_mlp(g3, wa1, ba1, wc, cc):
    return pl.pallas_call(
        _angle_body,
        grid=(T // BT,),
        in_specs=[
            pl.BlockSpec((3, BT), lambda i: (0, i)),
            pl.BlockSpec((3, NB), lambda i: (0, 0)),
            pl.BlockSpec((1, NB), lambda i: (0, 0)),
            pl.BlockSpec((NB, D), lambda i: (0, 0)),
            pl.BlockSpec((1, D), lambda i: (0, 0)),
        ],
        out_specs=pl.BlockSpec((BT, D), lambda i: (i, 0)),
        out_shape=jax.ShapeDtypeStruct((T, D), jnp.float32),
    )(g3, wa1, ba1, wc, cc)


def _final_body(s_ref, cnt_ref, m_ref, b2u_ref, bu_ref, o_ref):
    s = s_ref[...]
    cnt = cnt_ref[...]
    o_ref[...] = (jnp.dot(s, m_ref[...], preferred_element_type=jnp.float32)
                  + cnt * b2u_ref[...] + bu_ref[...])


def _final_matmul(s, cnt2d, m, b2u, bu):
    return pl.pallas_call(
        _final_body,
        grid=(E // BE,),
        in_specs=[
            pl.BlockSpec((BE, D), lambda i: (i, 0)),
            pl.BlockSpec((BE, 1), lambda i: (i, 0)),
            pl.BlockSpec((D, D), lambda i: (0, 0)),
            pl.BlockSpec((1, D), lambda i: (0, 0)),
            pl.BlockSpec((1, D), lambda i: (0, 0)),
        ],
        out_specs=pl.BlockSpec((BE, D), lambda i: (i, 0)),
        out_shape=jax.ShapeDtypeStruct((E, D), jnp.float32),
    )(s, cnt2d, m, b2u, bu)


NW = 32
TPW = T // NW
GCH = 4000


def _geo_body(eij_hbm, eik_hbm, vx_hbm, vy_hbm, vz_hbm,
              g0_hbm, g1_hbm, g2_hbm,
              iij, iik, xij, yij, zij, xik, yik, zik, g0, g1, g2, sem):
    wid = lax.axis_index("s") * 2 + lax.axis_index("c")

    for ch in range(TPW // GCH):
        base = wid * TPW + ch * GCH
        pltpu.sync_copy(eij_hbm.at[pl.ds(base, GCH)], iij)
        pltpu.sync_copy(eik_hbm.at[pl.ds(base, GCH)], iik)
        cps = [pltpu.async_copy(vx_hbm.at[iij], xij, sem),
               pltpu.async_copy(vy_hbm.at[iij], yij, sem),
               pltpu.async_copy(vz_hbm.at[iij], zij, sem),
               pltpu.async_copy(vx_hbm.at[iik], xik, sem),
               pltpu.async_copy(vy_hbm.at[iik], yik, sem),
               pltpu.async_copy(vz_hbm.at[iik], zik, sem)]
        for cp in cps:
            cp.wait()

        def body(i, carry):
            s = pl.ds(i * 16, 16)
            a, b, c = xij[s], yij[s], zij[s]
            d, e, f = xik[s], yik[s], zik[s]
            g0[s] = a * a + b * b + c * c
            g1[s] = d * d + e * e + f * f
            g2[s] = a * d + b * e + c * f
            return carry

        lax.fori_loop(0, GCH // 16, body, 0)
        pltpu.sync_copy(g0, g0_hbm.at[pl.ds(base, GCH)])
        pltpu.sync_copy(g1, g1_hbm.at[pl.ds(base, GCH)])
        pltpu.sync_copy(g2, g2_hbm.at[pl.ds(base, GCH)])


def _geometry(e_ij, e_ik, vx, vy, vz):
    mesh = plsc.VectorSubcoreMesh(core_axis_name="c", subcore_axis_name="s")
    f = pl.kernel(
        _geo_body,
        out_type=(jax.ShapeDtypeStruct((T,), jnp.float32),
                  jax.ShapeDtypeStruct((T,), jnp.float32),
                  jax.ShapeDtypeStruct((T,), jnp.float32)),
        mesh=mesh,
        scratch_types=(
            [pltpu.VMEM((GCH,), jnp.int32)] * 2
            + [pltpu.VMEM((GCH,), jnp.float32)] * 9
            + [pltpu.SemaphoreType.DMA]
        ),
    )
    return f(e_ij, e_ik, vx, vy, vz)


BROW = 8192
NBK = 40
EP = BROW * NBK
ACC = BROW + 16
DUMMY = BROW
SCH = 4000
CAP = SCH // 16
SUBSZ = 16 * CAP + 32
COMPSZ = SCH + 80
TPS = T // 16
CSUB = 64


def _silu16(x):
    return x / (1.0 + jnp.exp(-x))


def _scatter_body(eij_hbm, eik_hbm, p_hbm, q_hbm, a_hbm,
                  s_hbm, cnt_hbm,
                  ebij, ebik, tsub, ijsub, iksub, tcomp, ijcomp, ikcomp,
                  pbuf, qbuf, abuf, lidx, lidx16, ones_c, zrow, zcnt,
                  accum, cntacc, sem):
    cid = lax.axis_index("c")
    tid = lax.axis_index("s")
    lane = lax.iota(jnp.int32, 16)
    lanecap = lane * CAP
    zvec = jnp.zeros((16,), jnp.int32)

    def init_body(i, carry):
        sl = pl.ds(i * 16, 16)
        tsub[sl] = zvec
        ijsub[sl] = zvec
        iksub[sl] = zvec
        return carry
    lax.fori_loop(0, SUBSZ // 16, init_body, 0)

    def init_body2(i, carry):
        sl = pl.ds(i * 16, 16)
        tcomp[sl] = zvec
        ijcomp[sl] = zvec
        ikcomp[sl] = zvec
        return carry
    lax.fori_loop(0, COMPSZ // 16, init_body2, 0)

    def zrow_body(i, carry):
        for k in range(D // 16):
            zrow[i, pl.ds(k * 16, 16)] = jnp.zeros((16,), jnp.float32)
        return carry
    lax.fori_loop(0, 32, zrow_body, 0)

    def zcnt_body(i, carry):
        zcnt[pl.ds(i * 16, 16)] = jnp.zeros((16,), jnp.float32)
        return carry
    lax.fori_loop(0, 512 // 16, zcnt_body, 0)

    def ones_body(i, carry):
        ones_c[pl.ds(i * 16, 16)] = jnp.ones((16,), jnp.float32)
        return carry
    lax.fori_loop(0, CSUB // 16, ones_body, 0)

    def bucket_body(j, carry0):
        p = 2 * j + cid
        lo = p * BROW

        for r in range(16):
            pltpu.sync_copy(zrow, accum.at[pl.ds(tid * 512 + r * 32, 32)])
        pltpu.sync_copy(zcnt, cntacc.at[pl.ds(tid * 512, 512)])

        @pl.when(tid == 0)
        def _():
            pltpu.sync_copy(zcnt.at[pl.ds(0, 16)],
                            cntacc.at[pl.ds(BROW, 16)])
            pltpu.sync_copy(zrow.at[pl.ds(0, 16)],
                            accum.at[pl.ds(BROW, 16)])

        plsc.subcore_barrier()

        def chunk_body(sc, carry1):
            sbase = tid * TPS + sc * SCH
            pltpu.sync_copy(eij_hbm.at[pl.ds(sbase, SCH)], ebij)
            pltpu.sync_copy(eik_hbm.at[pl.ds(sbase, SCH)], ebik)

            def cbody(i, w_vec):
                sl = pl.ds(i * 16, 16)
                ev = ebij[sl]
                kv = ebik[sl]
                m = (ev >= lo) & (ev < lo + BROW)
                mi = jnp.where(m, 1, 0)
                tv = lane + (sbase + i * 16)
                pos = jnp.where(m, lanecap + w_vec, (16 * CAP + 16) + lane)
                plsc.store_scatter(tsub, [pos], tv)
                plsc.store_scatter(ijsub, [pos], ev)
                plsc.store_scatter(iksub, [pos], kv)
                return w_vec + mi

            counts = lax.fori_loop(0, SCH // 16, cbody, zvec)

            m_tot = jnp.int32(0)
            for l in range(16):
                cl = counts[l]
                ng = (cl + 15) // 16
                mt = m_tot

                def copyg(g, carry2, _l=l, _mt=mt):
                    d = pl.ds(_mt + g * 16, 16)
                    s = pl.ds(_l * CAP + g * 16, 16)
                    tcomp[d] = tsub[s]
                    ijcomp[d] = ijsub[s]
                    ikcomp[d] = iksub[s]
                    return carry2
                lax.fori_loop(0, ng, copyg, 0)
                m_tot = m_tot + cl

            nfull = m_tot // CSUB
            ntail = (m_tot - nfull * CSUB + 15) // 16

            def pbody(u, carry2):
                off = u * CSUB
                ij_c = ijcomp.at[pl.ds(off, CSUB)]
                ik_c = ikcomp.at[pl.ds(off, CSUB)]
                t_c = tcomp.at[pl.ds(off, CSUB)]
                cp1 = pltpu.async_copy(p_hbm.at[ij_c], pbuf, sem)
                cp2 = pltpu.async_copy(q_hbm.at[ik_c], qbuf, sem)
                cp3 = pltpu.async_copy(a_hbm.at[t_c], abuf, sem)
                cp1.wait()
                cp2.wait()
                cp3.wait()

                def zbody(r, carry3):
                    for k in range(D // 16):
                        sl = pl.ds(k * 16, 16)
                        x = pbuf[r, sl] + qbuf[r, sl] + abuf[r, sl]
                        pbuf[r, sl] = _silu16(x)
                    return carry3
                lax.fori_loop(0, CSUB, zbody, 0)

                def lbody(i, carry3):
                    sl = pl.ds(i * 16, 16)
                    e = ijcomp[pl.ds(off + i * 16, 16)]
                    lidx[sl] = e - lo
                    return carry3
                lax.fori_loop(0, CSUB // 16, lbody, 0)

                pltpu.sync_copy(pbuf, accum.at[lidx], add=True)
                pltpu.sync_copy(ones_c, cntacc.at[lidx], add=True)
                return carry2

            lax.fori_loop(0, nfull, pbody, 0)

            def tbody(tt, carry2):
                off = nfull * CSUB + tt * 16
                sl16 = pl.ds(0, 16)
                ij_c = ijcomp.at[pl.ds(off, 16)]
                ik_c = ikcomp.at[pl.ds(off, 16)]
                t_c = tcomp.at[pl.ds(off, 16)]
                pb = pbuf.at[sl16]
                qb = qbuf.at[sl16]
                ab = abuf.at[sl16]
                cp1 = pltpu.async_copy(p_hbm.at[ij_c], pb, sem)
                cp2 = pltpu.async_copy(q_hbm.at[ik_c], qb, sem)
                cp3 = pltpu.async_copy(a_hbm.at[t_c], ab, sem)
                cp1.wait()
                cp2.wait()
                cp3.wait()

                def zbody(r, carry3):
                    for k in range(D // 16):
                        sl = pl.ds(k * 16, 16)
                        x = pbuf[r, sl] + qbuf[r, sl] + abuf[r, sl]
                        pbuf[r, sl] = _silu16(x)
                    return carry3
                lax.fori_loop(0, 16, zbody, 0)

                e = ijcomp[pl.ds(off, 16)]
                posv = off + lane
                valid = posv < m_tot
                lidx16[pl.ds(0, 16)] = jnp.where(valid, e - lo, DUMMY)
                pltpu.sync_copy(pbuf.at[pl.ds(0, 16)],
                                accum.at[lidx16], add=True)
                pltpu.sync_copy(ones_c.at[pl.ds(0, 16)],
                                cntacc.at[lidx16], add=True)
                return carry2

            lax.fori_loop(0, ntail, tbody, 0)
            return carry1

        lax.fori_loop(0, TPS // SCH, chunk_body, 0)
        plsc.subcore_barrier()

        pltpu.sync_copy(accum.at[pl.ds(tid * 512, 512)],
                        s_hbm.at[pl.ds(lo + tid * 512, 512)])

        @pl.when(tid == 0)
        def _():
            pltpu.sync_copy(cntacc.at[pl.ds(0, BROW)],
                            cnt_hbm.at[pl.ds(lo, BROW)])
        return carry0

    lax.fori_loop(0, NBK // 2, bucket_body, 0)


def _gather_silu_scatter(e_ij, e_ik, p_arr, q_arr, a_arr):
    mesh = plsc.VectorSubcoreMesh(core_axis_name="c", subcore_axis_name="s")
    f = pl.kernel(
        _scatter_body,
        out_type=(jax.ShapeDtypeStruct((EP, D), jnp.float32),
                  jax.ShapeDtypeStruct((EP,), jnp.float32)),
        mesh=mesh,
        compiler_params=pltpu.CompilerParams(needs_layout_passes=False),
        scratch_types=[
            pltpu.VMEM((SCH,), jnp.int32),
            pltpu.VMEM((SCH,), jnp.int32),
            pltpu.VMEM((SUBSZ,), jnp.int32),
            pltpu.VMEM((SUBSZ,), jnp.int32),
            pltpu.VMEM((SUBSZ,), jnp.int32),
            pltpu.VMEM((COMPSZ,), jnp.int32),
            pltpu.VMEM((COMPSZ,), jnp.int32),
            pltpu.VMEM((COMPSZ,), jnp.int32),
            pltpu.VMEM((CSUB, D), jnp.float32),
            pltpu.VMEM((CSUB, D), jnp.float32),
            pltpu.VMEM((CSUB, D), jnp.float32),
            pltpu.VMEM((CSUB,), jnp.int32),
            pltpu.VMEM((16,), jnp.int32),
            pltpu.VMEM((CSUB,), jnp.float32),
            pltpu.VMEM((32, D), jnp.float32),
            pltpu.VMEM((512,), jnp.float32),
            pltpu.VMEM_SHARED((ACC, D), jnp.float32),
            pltpu.VMEM_SHARED((ACC,), jnp.float32),
            pltpu.SemaphoreType.DMA,
        ],
    )
    return f(e_ij, e_ik, p_arr, q_arr, a_arr)


def kernel(edge_attr, three_body_indices, three_body_edge_indices,
           edge_vectors, Wa1, ba1, Wa2, ba2, W1, b1, W2, b2, Wu, bu):
    del three_body_indices
    e_ij = three_body_edge_indices[:, 0]
    e_ik = three_body_edge_indices[:, 1]

    w_ab = W1[:2 * D, :]
    w_ab = jnp.concatenate([W1[:D, :], W1[D:2 * D, :]], axis=1)
    wc = Wa2 @ W1[2 * D:, :]
    cc = (ba2 @ W1[2 * D:, :] + b1)[None, :]
    m_fold = W2 @ Wu
    b2u = (b2 @ Wu)[None, :]
    bu2 = bu[None, :]

    p_arr, q_arr = _pq_matmul(edge_attr, w_ab)

    vx = edge_vectors[:, 0]
    vy = edge_vectors[:, 1]
    vz = edge_vectors[:, 2]
    g0, g1, g2 = _geometry(e_ij, e_ik, vx, vy, vz)
    g3 = jnp.stack([g0, g1, g2], axis=0)

    a_arr = _angle_mlp(g3, Wa1, ba1[None, :], wc, cc)

    s_pad, cnt_pad = _gather_silu_scatter(e_ij, e_ik, p_arr, q_arr, a_arr)

    return _final_matmul(s_pad, cnt_pad[:, None], m_fold, b2u, bu2)
